# baseline (speedup 1.0000x reference)
import functools

import jax
import jax.numpy as jnp
from jax import lax
from jax.experimental import pallas as pl
from jax.experimental.pallas import tpu as pltpu
from jax.experimental.pallas import tpu_sc as plsc

B = 16
V0, V1, V2, V3 = 50000, 12500, 3125, 781
L = 9
C_IN = 3
H1, H2, H3 = 16, 16, 32
LATENT = 128

NC, NS = 2, 16
NW = NC * NS
LANES = 16


def _cdiv(a, b):
    return (a + b - 1) // b


def _sc_gather(table, idx2d, d, tc_tiling=False):
    nchunk, g = idx2d.shape
    dt = table.dtype
    iters = _cdiv(nchunk, NW)
    mesh = plsc.VectorSubcoreMesh(core_axis_name="c", subcore_axis_name="s")

    @functools.partial(
        pl.kernel,
        mesh=mesh,
        out_type=jax.ShapeDtypeStruct((nchunk, g, d), dt),
        scratch_types=[
            pltpu.VMEM((g,), jnp.int32),
            pltpu.VMEM((g, d), dt),
            pltpu.SemaphoreType.DMA,
        ],
        compiler_params=pltpu.CompilerParams(use_tc_tiling_on_sc=tc_tiling, needs_layout_passes=False),
    )
    def k(table_hbm, idx_hbm, out_hbm, idx_v, rows_v, sem):
        wid = lax.axis_index("s") * NC + lax.axis_index("c")

        def body(t, carry):
            chunk = t * NW + wid

            @pl.when(chunk < nchunk)
            def _():
                pltpu.sync_copy(idx_hbm.at[chunk], idx_v)
                pltpu.async_copy(table_hbm.at[idx_v], rows_v, sem).wait()
                pltpu.sync_copy(rows_v, out_hbm.at[chunk])

            return carry

        lax.fori_loop(0, iters, body, 0)

    return k(table, idx2d)


def _sc_pool(table, idx2d, w2d, u_per_chunk, d, tc_tiling=False):
    nchunk, gsz = idx2d.shape
    iters = _cdiv(nchunk, NW)
    nseg = d // LANES
    mesh = plsc.VectorSubcoreMesh(core_axis_name="c", subcore_axis_name="s")

    @functools.partial(
        pl.kernel,
        mesh=mesh,
        out_type=jax.ShapeDtypeStruct((nchunk, u_per_chunk, d), jnp.float32),
        scratch_types=[
            pltpu.VMEM((gsz,), jnp.int32),
            pltpu.VMEM((gsz,), jnp.float32),
            pltpu.VMEM((gsz, d), jnp.float32),
            pltpu.VMEM((u_per_chunk, d), jnp.float32),
            pltpu.SemaphoreType.DMA,
        ],
        compiler_params=pltpu.CompilerParams(use_tc_tiling_on_sc=tc_tiling, needs_layout_passes=False),
    )
    def k(table_hbm, idx_hbm, w_hbm, out_hbm, idx_v, w_v, g_v, o_v, sem):
        wid = lax.axis_index("s") * NC + lax.axis_index("c")

        def body(t, carry):
            chunk = t * NW + wid

            @pl.when(chunk < nchunk)
            def _():
                pltpu.sync_copy(idx_hbm.at[chunk], idx_v)
                pltpu.sync_copy(w_hbm.at[chunk], w_v)
                pltpu.async_copy(table_hbm.at[idx_v], g_v, sem).wait()

                def ubody(u, c2):
                    r0 = u * 3
                    wv = [
                        plsc.load_gather(
                            w_v, [jnp.full((LANES,), r0 + kk, jnp.int32)]
                        )
                        for kk in range(3)
                    ]
                    for j in range(nseg):
                        sl = pl.ds(j * LANES, LANES)
                        acc = wv[0] * g_v[r0, sl]
                        acc = acc + wv[1] * g_v[r0 + 1, sl]
                        acc = acc + wv[2] * g_v[r0 + 2, sl]
                        o_v[u, sl] = acc
                    return c2

                lax.fori_loop(0, u_per_chunk, ubody, 0)
                pltpu.sync_copy(o_v, out_hbm.at[chunk])

            return carry

        lax.fori_loop(0, iters, body, 0)

    return k(table, idx2d, w2d)


def _sc_pool3_t(table, idx2d, w2d, u_per_chunk, tc_tiling=False):
    nchunk, gsz = idx2d.shape
    d = B * H3
    hseg = H3 // LANES
    iters = _cdiv(nchunk, NW)
    mesh = plsc.VectorSubcoreMesh(core_axis_name="c", subcore_axis_name="s")

    @functools.partial(
        pl.kernel,
        mesh=mesh,
        out_type=jax.ShapeDtypeStruct(
            (B, nchunk, u_per_chunk * H3), jnp.float32
        ),
        scratch_types=[
            pltpu.VMEM((gsz,), jnp.int32),
            pltpu.VMEM((gsz,), jnp.float32),
            pltpu.VMEM((gsz, d), jnp.float32),
            pltpu.VMEM((B, u_per_chunk * H3), jnp.float32),
            pltpu.SemaphoreType.DMA,
        ],
        compiler_params=pltpu.CompilerParams(use_tc_tiling_on_sc=tc_tiling, needs_layout_passes=False),
    )
    def k(table_hbm, idx_hbm, w_hbm, out_hbm, idx_v, w_v, g_v, o_v, sem):
        wid = lax.axis_index("s") * NC + lax.axis_index("c")

        def body(t, carry):
            chunk = t * NW + wid

            @pl.when(chunk < nchunk)
            def _():
                pltpu.sync_copy(idx_hbm.at[chunk], idx_v)
                pltpu.sync_copy(w_hbm.at[chunk], w_v)
                pltpu.async_copy(table_hbm.at[idx_v], g_v, sem).wait()

                def ubody(u, c2):
                    r0 = u * 3
                    wv = [
                        plsc.load_gather(
                            w_v, [jnp.full((LANES,), r0 + kk, jnp.int32)]
                        )
                        for kk in range(3)
                    ]
                    for bb in range(B):
                        for j in range(hseg):
                            src = pl.ds(bb * H3 + j * LANES, LANES)
                            acc = wv[0] * g_v[r0, src]
                            acc = acc + wv[1] * g_v[r0 + 1, src]
                            acc = acc + wv[2] * g_v[r0 + 2, src]
                            o_v[bb, pl.ds(u * H3 + j * LANES, LANES)] = acc
                    return c2

                lax.fori_loop(0, u_per_chunk, ubody, 0)
                for bb in range(B):
                    pltpu.sync_copy(o_v.at[bb], out_hbm.at[bb, chunk])

            return carry

        lax.fori_loop(0, iters, body, 0)

    return k(table, idx2d, w2d)


def _tc_mm(a, w, bias, bm, elu):
    m, kdim = a.shape
    _, n = w.shape

    def body(a_ref, w_ref, b_ref, o_ref):
        y = jnp.dot(a_ref[...], w_ref[...], preferred_element_type=jnp.float32)
        y = y + b_ref[...]
        if elu:
            y = jnp.where(y > 0.0, y, jnp.exp(jnp.minimum(y, 0.0)) - 1.0)
        o_ref[...] = y

    return pl.pallas_call(
        body,
        grid=(_cdiv(m, bm),),
        in_specs=[
            pl.BlockSpec((bm, kdim), lambda i: (i, 0)),
            pl.BlockSpec((kdim, n), lambda i: (0, 0)),
            pl.BlockSpec((1, n), lambda i: (0, 0)),
        ],
        out_specs=pl.BlockSpec((bm, n), lambda i: (i, 0)),
        out_shape=jax.ShapeDtypeStruct((m, n), jnp.float32),
    )(a, w, bias.reshape(1, n))


def _expand_w(w, l, cin, h):
    eye = jnp.eye(B, dtype=jnp.float32)
    wp = jnp.einsum("lch,bd->lbcdh", w.reshape(l, cin, h), eye)
    return wp.reshape(l * B * cin, B * h)


def kernel(x, spiral1, spiral2, spiral3, pool1_idx, pool2_idx, pool3_idx,
           pool1_w, pool2_w, pool3_w, W1, b1, W2, b2, W3, b3, Wp, bp):
    x0 = jnp.transpose(x, (1, 0, 2)).reshape(V0, B * C_IN).astype(jnp.bfloat16)

    w1p = _expand_w(W1, L, C_IN, H1)
    w2p = _expand_w(W2, L, H1, H2)
    w3p = _expand_w(W3, L, H2, H3)
    b1p = jnp.tile(b1, B)
    b2p = jnp.tile(b2, B)
    b3p = jnp.tile(b3, B)

    a1 = _sc_gather(x0, spiral1.reshape(3600, 125), B * C_IN, tc_tiling=False)
    h1 = _tc_mm(a1.reshape(V0, L * B * C_IN), w1p.astype(jnp.bfloat16),
                b1p, bm=1024, elu=True)
    p1 = _sc_pool(h1, pool1_idx.reshape(500, 75),
                  pool1_w.reshape(500, 75), 25, B * H1)

    a2 = _sc_gather(p1.reshape(V1, B * H1).astype(jnp.bfloat16),
                    spiral2.reshape(900, 125), B * H1)
    h2 = _tc_mm(a2.reshape(V1, L * B * H1), w2p.astype(jnp.bfloat16),
                b2p, bm=512, elu=True)
    p2 = _sc_pool(h2, pool2_idx.reshape(125, 75),
                  pool2_w.reshape(125, 75), 25, B * H2)

    a3 = _sc_gather(p2.reshape(V2, B * H2).astype(jnp.bfloat16),
                    spiral3.reshape(225, 125), B * H2)
    h3 = _tc_mm(a3.reshape(V2, L * B * H2), w3p.astype(jnp.bfloat16),
                b3p, bm=512, elu=True)
    qt = _sc_pool3_t(h3, pool3_idx.reshape(71, 33),
                     pool3_w.reshape(71, 33), 11)

    z = _tc_mm(qt.reshape(B, V3 * H3), Wp, bp, bm=16, elu=False)
    return z


# speedup vs baseline: 1.1446x; 1.1446x over previous
import functools

import jax
import jax.numpy as jnp
from jax import lax
from jax.experimental import pallas as pl
from jax.experimental.pallas import tpu as pltpu
from jax.experimental.pallas import tpu_sc as plsc

B = 16
V0, V1, V2, V3 = 50000, 12500, 3125, 781
L = 9
C_IN = 3
H1, H2, H3 = 16, 16, 32
LATENT = 128

NC, NS = 2, 16
NW = NC * NS
LANES = 16


def _cdiv(a, b):
    return (a + b - 1) // b


def _sc_gather(table, idx2d, d, tc_tiling=False):
    nchunk, g = idx2d.shape
    dt = table.dtype
    iters = _cdiv(nchunk, NW)
    mesh = plsc.VectorSubcoreMesh(core_axis_name="c", subcore_axis_name="s")

    @functools.partial(
        pl.kernel,
        mesh=mesh,
        out_type=jax.ShapeDtypeStruct((nchunk, g, d), dt),
        scratch_types=[
            pltpu.VMEM((g,), jnp.int32),
            pltpu.VMEM((g, d), dt),
            pltpu.SemaphoreType.DMA,
        ],
        compiler_params=pltpu.CompilerParams(use_tc_tiling_on_sc=tc_tiling, needs_layout_passes=False),
    )
    def k(table_hbm, idx_hbm, out_hbm, idx_v, rows_v, sem):
        wid = lax.axis_index("s") * NC + lax.axis_index("c")

        def body(t, carry):
            chunk = t * NW + wid

            @pl.when(chunk < nchunk)
            def _():
                pltpu.sync_copy(idx_hbm.at[chunk], idx_v)
                pltpu.async_copy(table_hbm.at[idx_v], rows_v, sem).wait()
                pltpu.sync_copy(rows_v, out_hbm.at[chunk])

            return carry

        lax.fori_loop(0, iters, body, 0)

    return k(table, idx2d)


def _sc_pool(table, idx2d, w2d, u_per_chunk, d, tc_tiling=False):
    nchunk, gsz = idx2d.shape
    iters = _cdiv(nchunk, NW)
    nseg = d // LANES
    mesh = plsc.VectorSubcoreMesh(core_axis_name="c", subcore_axis_name="s")

    @functools.partial(
        pl.kernel,
        mesh=mesh,
        out_type=jax.ShapeDtypeStruct((nchunk, u_per_chunk, d), jnp.float32),
        scratch_types=[
            pltpu.VMEM((gsz,), jnp.int32),
            pltpu.VMEM((gsz,), jnp.float32),
            pltpu.VMEM((gsz, d), jnp.float32),
            pltpu.VMEM((u_per_chunk, d), jnp.float32),
            pltpu.SemaphoreType.DMA,
        ],
        compiler_params=pltpu.CompilerParams(use_tc_tiling_on_sc=tc_tiling, needs_layout_passes=False),
    )
    def k(table_hbm, idx_hbm, w_hbm, out_hbm, idx_v, w_v, g_v, o_v, sem):
        wid = lax.axis_index("s") * NC + lax.axis_index("c")

        def body(t, carry):
            chunk = t * NW + wid

            @pl.when(chunk < nchunk)
            def _():
                pltpu.sync_copy(idx_hbm.at[chunk], idx_v)
                pltpu.sync_copy(w_hbm.at[chunk], w_v)
                pltpu.async_copy(table_hbm.at[idx_v], g_v, sem).wait()

                def ubody(u, c2):
                    r0 = u * 3
                    wv = [
                        plsc.load_gather(
                            w_v, [jnp.full((LANES,), r0 + kk, jnp.int32)]
                        )
                        for kk in range(3)
                    ]
                    for j in range(nseg):
                        sl = pl.ds(j * LANES, LANES)
                        acc = wv[0] * g_v[r0, sl]
                        acc = acc + wv[1] * g_v[r0 + 1, sl]
                        acc = acc + wv[2] * g_v[r0 + 2, sl]
                        o_v[u, sl] = acc
                    return c2

                lax.fori_loop(0, u_per_chunk, ubody, 0)
                pltpu.sync_copy(o_v, out_hbm.at[chunk])

            return carry

        lax.fori_loop(0, iters, body, 0)

    return k(table, idx2d, w2d)


def _sc_pool3_t(table, idx2d, w2d, u_per_chunk, tc_tiling=False):
    nchunk, gsz = idx2d.shape
    d = B * H3
    hseg = H3 // LANES
    iters = _cdiv(nchunk, NW)
    mesh = plsc.VectorSubcoreMesh(core_axis_name="c", subcore_axis_name="s")

    @functools.partial(
        pl.kernel,
        mesh=mesh,
        out_type=jax.ShapeDtypeStruct(
            (B, nchunk, u_per_chunk * H3), jnp.float32
        ),
        scratch_types=[
            pltpu.VMEM((gsz,), jnp.int32),
            pltpu.VMEM((gsz,), jnp.float32),
            pltpu.VMEM((gsz, d), jnp.float32),
            pltpu.VMEM((B, u_per_chunk * H3), jnp.float32),
            pltpu.SemaphoreType.DMA,
        ],
        compiler_params=pltpu.CompilerParams(use_tc_tiling_on_sc=tc_tiling, needs_layout_passes=False),
    )
    def k(table_hbm, idx_hbm, w_hbm, out_hbm, idx_v, w_v, g_v, o_v, sem):
        wid = lax.axis_index("s") * NC + lax.axis_index("c")

        def body(t, carry):
            chunk = t * NW + wid

            @pl.when(chunk < nchunk)
            def _():
                pltpu.sync_copy(idx_hbm.at[chunk], idx_v)
                pltpu.sync_copy(w_hbm.at[chunk], w_v)
                pltpu.async_copy(table_hbm.at[idx_v], g_v, sem).wait()

                def ubody(u, c2):
                    r0 = u * 3
                    wv = [
                        plsc.load_gather(
                            w_v, [jnp.full((LANES,), r0 + kk, jnp.int32)]
                        )
                        for kk in range(3)
                    ]
                    for bb in range(B):
                        for j in range(hseg):
                            src = pl.ds(bb * H3 + j * LANES, LANES)
                            acc = wv[0] * g_v[r0, src]
                            acc = acc + wv[1] * g_v[r0 + 1, src]
                            acc = acc + wv[2] * g_v[r0 + 2, src]
                            o_v[bb, pl.ds(u * H3 + j * LANES, LANES)] = acc
                    return c2

                lax.fori_loop(0, u_per_chunk, ubody, 0)
                for bb in range(B):
                    pltpu.sync_copy(o_v.at[bb], out_hbm.at[bb, chunk])

            return carry

        lax.fori_loop(0, iters, body, 0)

    return k(table, idx2d, w2d)


def _tc_mm(a, w, bias, bm, elu):
    m, kdim = a.shape
    _, n = w.shape

    def body(a_ref, w_ref, b_ref, o_ref):
        y = jnp.dot(a_ref[...], w_ref[...], preferred_element_type=jnp.float32)
        y = y + b_ref[...]
        if elu:
            y = jnp.where(y > 0.0, y, jnp.exp(jnp.minimum(y, 0.0)) - 1.0)
        o_ref[...] = y

    return pl.pallas_call(
        body,
        grid=(_cdiv(m, bm),),
        in_specs=[
            pl.BlockSpec((bm, kdim), lambda i: (i, 0)),
            pl.BlockSpec((kdim, n), lambda i: (0, 0)),
            pl.BlockSpec((1, n), lambda i: (0, 0)),
        ],
        out_specs=pl.BlockSpec((bm, n), lambda i: (i, 0)),
        out_shape=jax.ShapeDtypeStruct((m, n), jnp.float32),
    )(a, w, bias.reshape(1, n))


def _expand_w(w, l, cin, h):
    eye = jnp.eye(B, dtype=jnp.float32)
    wp = jnp.einsum("lch,bd->lbcdh", w.reshape(l, cin, h), eye)
    return wp.reshape(l * B * cin, B * h)


def kernel(x, spiral1, spiral2, spiral3, pool1_idx, pool2_idx, pool3_idx,
           pool1_w, pool2_w, pool3_w, W1, b1, W2, b2, W3, b3, Wp, bp):
    x0 = jnp.transpose(x, (1, 0, 2)).reshape(V0, B * C_IN).astype(jnp.bfloat16)

    w1p = _expand_w(W1, L, C_IN, H1)
    w2p = _expand_w(W2, L, H1, H2)
    w3p = _expand_w(W3, L, H2, H3)
    b1p = jnp.tile(b1, B)
    b2p = jnp.tile(b2, B)
    b3p = jnp.tile(b3, B)

    a1 = _sc_gather(x0, spiral1.reshape(3600, 125), B * C_IN, tc_tiling=False)
    h1 = _tc_mm(a1.reshape(V0, L * B * C_IN), w1p.astype(jnp.bfloat16),
                b1p, bm=1024, elu=True)
    p1 = _sc_pool(h1, pool1_idx.reshape(500, 75),
                  pool1_w.reshape(500, 75), 25, B * H1)

    a2 = _sc_gather(p1.reshape(V1, B * H1), spiral2.reshape(900, 125),
                    B * H1)
    h2 = _tc_mm(a2.reshape(V1, L * B * H1), w2p, b2p, bm=512, elu=True)
    p2 = _sc_pool(h2, pool2_idx.reshape(125, 75),
                  pool2_w.reshape(125, 75), 25, B * H2)

    a3 = _sc_gather(p2.reshape(V2, B * H2), spiral3.reshape(225, 125),
                    B * H2)
    h3 = _tc_mm(a3.reshape(V2, L * B * H2), w3p, b3p, bm=512, elu=True)
    qt = _sc_pool3_t(h3, pool3_idx.reshape(71, 33),
                     pool3_w.reshape(71, 33), 11)

    z = _tc_mm(qt.reshape(B, V3 * H3), Wp, bp, bm=16, elu=False)
    return z


# speedup vs baseline: 1.3378x; 1.1688x over previous
import functools

import jax
import jax.numpy as jnp
from jax import lax
from jax.experimental import pallas as pl
from jax.experimental.pallas import tpu as pltpu
from jax.experimental.pallas import tpu_sc as plsc

B = 16
V0, V1, V2, V3 = 50000, 12500, 3125, 781
L = 9
C_IN = 3
H1, H2, H3 = 16, 16, 32
LATENT = 128

NC, NS = 2, 16
NW = NC * NS
LANES = 16


def _cdiv(a, b):
    return (a + b - 1) // b


def _sc_gather(table, idx2d, d, tc_tiling=False):
    nchunk, g = idx2d.shape
    dt = table.dtype
    iters = _cdiv(nchunk, NW)
    npair = _cdiv(iters, 2)
    mesh = plsc.VectorSubcoreMesh(core_axis_name="c", subcore_axis_name="s")

    @functools.partial(
        pl.kernel,
        mesh=mesh,
        out_type=jax.ShapeDtypeStruct((nchunk, g, d), dt),
        scratch_types=[
            pltpu.VMEM((g,), jnp.int32),
            pltpu.VMEM((g,), jnp.int32),
            pltpu.VMEM((g, d), dt),
            pltpu.VMEM((g, d), dt),
            pltpu.SemaphoreType.DMA,
            pltpu.SemaphoreType.DMA,
        ],
        compiler_params=pltpu.CompilerParams(use_tc_tiling_on_sc=tc_tiling, needs_layout_passes=False),
    )
    def k(table_hbm, idx_hbm, out_hbm, idx_v0, idx_v1, rows_v0, rows_v1,
          sem0, sem1):
        wid = lax.axis_index("s") * NC + lax.axis_index("c")
        idx_b = (idx_v0, idx_v1)
        rows_b = (rows_v0, rows_v1)
        sem_b = (sem0, sem1)

        def fire(step, p):
            chunk = step * NW + wid

            @pl.when(chunk < nchunk)
            def _():
                pltpu.sync_copy(idx_hbm.at[chunk], idx_b[p])
                pltpu.make_async_copy(
                    table_hbm.at[idx_b[p]], rows_b[p], sem_b[p]
                ).start()

        def drain(step, p):
            chunk = step * NW + wid

            @pl.when(chunk < nchunk)
            def _():
                pltpu.make_async_copy(
                    table_hbm.at[idx_b[p]], rows_b[p], sem_b[p]
                ).wait()
                pltpu.sync_copy(rows_b[p], out_hbm.at[chunk])

        fire(0, 0)

        def body(t2, carry):
            t = t2 * 2
            fire(t + 1, 1)
            drain(t, 0)
            fire(t + 2, 0)
            drain(t + 1, 1)
            return carry

        lax.fori_loop(0, npair, body, 0)

    return k(table, idx2d)


def _sc_pool(table, idx2d, w2d, u_per_chunk, d, tc_tiling=False):
    nchunk, gsz = idx2d.shape
    iters = _cdiv(nchunk, NW)
    nseg = d // LANES
    mesh = plsc.VectorSubcoreMesh(core_axis_name="c", subcore_axis_name="s")

    @functools.partial(
        pl.kernel,
        mesh=mesh,
        out_type=jax.ShapeDtypeStruct((nchunk, u_per_chunk, d), jnp.float32),
        scratch_types=[
            pltpu.VMEM((gsz,), jnp.int32),
            pltpu.VMEM((gsz,), jnp.float32),
            pltpu.VMEM((gsz, d), jnp.float32),
            pltpu.VMEM((u_per_chunk, d), jnp.float32),
            pltpu.SemaphoreType.DMA,
        ],
        compiler_params=pltpu.CompilerParams(use_tc_tiling_on_sc=tc_tiling, needs_layout_passes=False),
    )
    def k(table_hbm, idx_hbm, w_hbm, out_hbm, idx_v, w_v, g_v, o_v, sem):
        wid = lax.axis_index("s") * NC + lax.axis_index("c")

        def body(t, carry):
            chunk = t * NW + wid

            @pl.when(chunk < nchunk)
            def _():
                pltpu.sync_copy(idx_hbm.at[chunk], idx_v)
                pltpu.sync_copy(w_hbm.at[chunk], w_v)
                pltpu.async_copy(table_hbm.at[idx_v], g_v, sem).wait()

                def ubody(u, c2):
                    r0 = u * 3
                    wv = [
                        plsc.load_gather(
                            w_v, [jnp.full((LANES,), r0 + kk, jnp.int32)]
                        )
                        for kk in range(3)
                    ]
                    for j in range(nseg):
                        sl = pl.ds(j * LANES, LANES)
                        acc = wv[0] * g_v[r0, sl]
                        acc = acc + wv[1] * g_v[r0 + 1, sl]
                        acc = acc + wv[2] * g_v[r0 + 2, sl]
                        o_v[u, sl] = acc
                    return c2

                lax.fori_loop(0, u_per_chunk, ubody, 0)
                pltpu.sync_copy(o_v, out_hbm.at[chunk])

            return carry

        lax.fori_loop(0, iters, body, 0)

    return k(table, idx2d, w2d)


def _sc_pool3_t(table, idx2d, w2d, u_per_chunk, tc_tiling=False):
    nchunk, gsz = idx2d.shape
    d = B * H3
    hseg = H3 // LANES
    iters = _cdiv(nchunk, NW)
    mesh = plsc.VectorSubcoreMesh(core_axis_name="c", subcore_axis_name="s")

    @functools.partial(
        pl.kernel,
        mesh=mesh,
        out_type=jax.ShapeDtypeStruct(
            (B, nchunk, u_per_chunk * H3), jnp.float32
        ),
        scratch_types=[
            pltpu.VMEM((gsz,), jnp.int32),
            pltpu.VMEM((gsz,), jnp.float32),
            pltpu.VMEM((gsz, d), jnp.float32),
            pltpu.VMEM((B, u_per_chunk * H3), jnp.float32),
            pltpu.SemaphoreType.DMA,
        ],
        compiler_params=pltpu.CompilerParams(use_tc_tiling_on_sc=tc_tiling, needs_layout_passes=False),
    )
    def k(table_hbm, idx_hbm, w_hbm, out_hbm, idx_v, w_v, g_v, o_v, sem):
        wid = lax.axis_index("s") * NC + lax.axis_index("c")

        def body(t, carry):
            chunk = t * NW + wid

            @pl.when(chunk < nchunk)
            def _():
                pltpu.sync_copy(idx_hbm.at[chunk], idx_v)
                pltpu.sync_copy(w_hbm.at[chunk], w_v)
                pltpu.async_copy(table_hbm.at[idx_v], g_v, sem).wait()

                def ubody(u, c2):
                    r0 = u * 3
                    wv = [
                        plsc.load_gather(
                            w_v, [jnp.full((LANES,), r0 + kk, jnp.int32)]
                        )
                        for kk in range(3)
                    ]
                    for bb in range(B):
                        for j in range(hseg):
                            src = pl.ds(bb * H3 + j * LANES, LANES)
                            acc = wv[0] * g_v[r0, src]
                            acc = acc + wv[1] * g_v[r0 + 1, src]
                            acc = acc + wv[2] * g_v[r0 + 2, src]
                            o_v[bb, pl.ds(u * H3 + j * LANES, LANES)] = acc
                    return c2

                lax.fori_loop(0, u_per_chunk, ubody, 0)
                for bb in range(B):
                    pltpu.sync_copy(o_v.at[bb], out_hbm.at[bb, chunk])

            return carry

        lax.fori_loop(0, iters, body, 0)

    return k(table, idx2d, w2d)


def _tc_mm(a, w, bias, bm, elu):
    m, kdim = a.shape
    _, n = w.shape

    def body(a_ref, w_ref, b_ref, o_ref):
        y = jnp.dot(a_ref[...], w_ref[...], preferred_element_type=jnp.float32)
        y = y + b_ref[...]
        if elu:
            y = jnp.where(y > 0.0, y, jnp.exp(jnp.minimum(y, 0.0)) - 1.0)
        o_ref[...] = y

    return pl.pallas_call(
        body,
        grid=(_cdiv(m, bm),),
        in_specs=[
            pl.BlockSpec((bm, kdim), lambda i: (i, 0)),
            pl.BlockSpec((kdim, n), lambda i: (0, 0)),
            pl.BlockSpec((1, n), lambda i: (0, 0)),
        ],
        out_specs=pl.BlockSpec((bm, n), lambda i: (i, 0)),
        out_shape=jax.ShapeDtypeStruct((m, n), jnp.float32),
    )(a, w, bias.reshape(1, n))


def _expand_w(w, l, cin, h):
    eye = jnp.eye(B, dtype=jnp.float32)
    wp = jnp.einsum("lch,bd->lbcdh", w.reshape(l, cin, h), eye)
    return wp.reshape(l * B * cin, B * h)


def kernel(x, spiral1, spiral2, spiral3, pool1_idx, pool2_idx, pool3_idx,
           pool1_w, pool2_w, pool3_w, W1, b1, W2, b2, W3, b3, Wp, bp):
    x0 = jnp.transpose(x, (1, 0, 2)).reshape(V0, B * C_IN)

    w1p = _expand_w(W1, L, C_IN, H1)
    w2p = _expand_w(W2, L, H1, H2)
    w3p = _expand_w(W3, L, H2, H3)
    b1p = jnp.tile(b1, B)
    b2p = jnp.tile(b2, B)
    b3p = jnp.tile(b3, B)

    a1 = _sc_gather(x0, spiral1.reshape(3600, 125), B * C_IN, tc_tiling=False)
    h1 = _tc_mm(a1.reshape(V0, L * B * C_IN), w1p, b1p, bm=1024, elu=True)
    p1 = _sc_pool(h1, pool1_idx.reshape(500, 75),
                  pool1_w.reshape(500, 75), 25, B * H1)

    a2 = _sc_gather(p1.reshape(V1, B * H1), spiral2.reshape(900, 125),
                    B * H1)
    h2 = _tc_mm(a2.reshape(V1, L * B * H1), w2p, b2p, bm=512, elu=True)
    p2 = _sc_pool(h2, pool2_idx.reshape(125, 75),
                  pool2_w.reshape(125, 75), 25, B * H2)

    a3 = _sc_gather(p2.reshape(V2, B * H2), spiral3.reshape(225, 125),
                    B * H2)
    h3 = _tc_mm(a3.reshape(V2, L * B * H2), w3p, b3p, bm=512, elu=True)
    qt = _sc_pool3_t(h3, pool3_idx.reshape(71, 33),
                     pool3_w.reshape(71, 33), 11)

    z = _tc_mm(qt.reshape(B, V3 * H3), Wp, bp, bm=16, elu=False)
    return z


# speedup vs baseline: 1.3768x; 1.0291x over previous
import functools

import jax
import jax.numpy as jnp
from jax import lax
from jax.experimental import pallas as pl
from jax.experimental.pallas import tpu as pltpu
from jax.experimental.pallas import tpu_sc as plsc

B = 16
V0, V1, V2, V3 = 50000, 12500, 3125, 781
L = 9
C_IN = 3
H1, H2, H3 = 16, 16, 32
LATENT = 128

NC, NS = 2, 16
NW = NC * NS
LANES = 16


def _cdiv(a, b):
    return (a + b - 1) // b


def _sc_gather(table, idx2d, d, tc_tiling=False):
    nchunk, g = idx2d.shape
    dt = table.dtype
    iters = _cdiv(nchunk, NW)
    npair = _cdiv(iters, 2)
    mesh = plsc.VectorSubcoreMesh(core_axis_name="c", subcore_axis_name="s")

    @functools.partial(
        pl.kernel,
        mesh=mesh,
        out_type=jax.ShapeDtypeStruct((nchunk, g, d), dt),
        scratch_types=[
            pltpu.VMEM((g,), jnp.int32),
            pltpu.VMEM((g,), jnp.int32),
            pltpu.VMEM((g, d), dt),
            pltpu.VMEM((g, d), dt),
            pltpu.SemaphoreType.DMA,
            pltpu.SemaphoreType.DMA,
        ],
        compiler_params=pltpu.CompilerParams(use_tc_tiling_on_sc=tc_tiling, needs_layout_passes=False),
    )
    def k(table_hbm, idx_hbm, out_hbm, idx_v0, idx_v1, rows_v0, rows_v1,
          sem0, sem1):
        wid = lax.axis_index("s") * NC + lax.axis_index("c")
        idx_b = (idx_v0, idx_v1)
        rows_b = (rows_v0, rows_v1)
        sem_b = (sem0, sem1)

        def fire(step, p):
            chunk = step * NW + wid

            @pl.when(chunk < nchunk)
            def _():
                pltpu.sync_copy(idx_hbm.at[chunk], idx_b[p])
                pltpu.make_async_copy(
                    table_hbm.at[idx_b[p]], rows_b[p], sem_b[p]
                ).start()

        def drain(step, p):
            chunk = step * NW + wid

            @pl.when(chunk < nchunk)
            def _():
                pltpu.make_async_copy(
                    table_hbm.at[idx_b[p]], rows_b[p], sem_b[p]
                ).wait()
                pltpu.sync_copy(rows_b[p], out_hbm.at[chunk])

        fire(0, 0)

        def body(t2, carry):
            t = t2 * 2
            fire(t + 1, 1)
            drain(t, 0)
            fire(t + 2, 0)
            drain(t + 1, 1)
            return carry

        lax.fori_loop(0, npair, body, 0)

    return k(table, idx2d)


def _sc_pool(table, idx2d, w2d, u_per_chunk, d, tc_tiling=False):
    nchunk, gsz = idx2d.shape
    iters = _cdiv(nchunk, NW)
    nseg = d // LANES
    mesh = plsc.VectorSubcoreMesh(core_axis_name="c", subcore_axis_name="s")

    @functools.partial(
        pl.kernel,
        mesh=mesh,
        out_type=jax.ShapeDtypeStruct((nchunk, u_per_chunk, d), jnp.float32),
        scratch_types=[
            pltpu.VMEM((gsz,), jnp.int32),
            pltpu.VMEM((gsz,), jnp.int32),
            pltpu.VMEM((gsz,), jnp.float32),
            pltpu.VMEM((gsz,), jnp.float32),
            pltpu.VMEM((gsz, d), jnp.float32),
            pltpu.VMEM((gsz, d), jnp.float32),
            pltpu.VMEM((u_per_chunk, d), jnp.float32),
            pltpu.SemaphoreType.DMA,
            pltpu.SemaphoreType.DMA,
        ],
        compiler_params=pltpu.CompilerParams(use_tc_tiling_on_sc=tc_tiling, needs_layout_passes=False),
    )
    def k(table_hbm, idx_hbm, w_hbm, out_hbm, idx_v0, idx_v1, w_v0, w_v1,
          g_v0, g_v1, o_v, sem0, sem1):
        wid = lax.axis_index("s") * NC + lax.axis_index("c")
        idx_b = (idx_v0, idx_v1)
        w_b = (w_v0, w_v1)
        g_b = (g_v0, g_v1)
        sem_b = (sem0, sem1)

        def fire(step, p):
            chunk = step * NW + wid

            @pl.when(chunk < nchunk)
            def _():
                pltpu.sync_copy(idx_hbm.at[chunk], idx_b[p])
                pltpu.sync_copy(w_hbm.at[chunk], w_b[p])
                pltpu.make_async_copy(
                    table_hbm.at[idx_b[p]], g_b[p], sem_b[p]
                ).start()

        def drain(step, p):
            chunk = step * NW + wid
            g_v, w_v = g_b[p], w_b[p]

            @pl.when(chunk < nchunk)
            def _():
                pltpu.make_async_copy(
                    table_hbm.at[idx_b[p]], g_v, sem_b[p]
                ).wait()

                def ubody(u, c2):
                    r0 = u * 3
                    wv = [
                        plsc.load_gather(
                            w_v, [jnp.full((LANES,), r0 + kk, jnp.int32)]
                        )
                        for kk in range(3)
                    ]
                    for j in range(nseg):
                        sl = pl.ds(j * LANES, LANES)
                        acc = wv[0] * g_v[r0, sl]
                        acc = acc + wv[1] * g_v[r0 + 1, sl]
                        acc = acc + wv[2] * g_v[r0 + 2, sl]
                        o_v[u, sl] = acc
                    return c2

                lax.fori_loop(0, u_per_chunk, ubody, 0)
                pltpu.sync_copy(o_v, out_hbm.at[chunk])

        fire(0, 0)

        def body(t2, carry):
            t = t2 * 2
            fire(t + 1, 1)
            drain(t, 0)
            fire(t + 2, 0)
            drain(t + 1, 1)
            return carry

        lax.fori_loop(0, _cdiv(iters, 2), body, 0)

    return k(table, idx2d, w2d)


def _sc_pool3_t(table, idx2d, w2d, u_per_chunk, tc_tiling=False):
    nchunk, gsz = idx2d.shape
    d = B * H3
    hseg = H3 // LANES
    iters = _cdiv(nchunk, NW)
    mesh = plsc.VectorSubcoreMesh(core_axis_name="c", subcore_axis_name="s")

    @functools.partial(
        pl.kernel,
        mesh=mesh,
        out_type=jax.ShapeDtypeStruct(
            (B, nchunk, u_per_chunk * H3), jnp.float32
        ),
        scratch_types=[
            pltpu.VMEM((gsz,), jnp.int32),
            pltpu.VMEM((gsz,), jnp.float32),
            pltpu.VMEM((gsz, d), jnp.float32),
            pltpu.VMEM((B, u_per_chunk * H3), jnp.float32),
            pltpu.SemaphoreType.DMA,
        ],
        compiler_params=pltpu.CompilerParams(use_tc_tiling_on_sc=tc_tiling, needs_layout_passes=False),
    )
    def k(table_hbm, idx_hbm, w_hbm, out_hbm, idx_v, w_v, g_v, o_v, sem):
        wid = lax.axis_index("s") * NC + lax.axis_index("c")

        def body(t, carry):
            chunk = t * NW + wid

            @pl.when(chunk < nchunk)
            def _():
                pltpu.sync_copy(idx_hbm.at[chunk], idx_v)
                pltpu.sync_copy(w_hbm.at[chunk], w_v)
                pltpu.async_copy(table_hbm.at[idx_v], g_v, sem).wait()

                def ubody(u, c2):
                    r0 = u * 3
                    wv = [
                        plsc.load_gather(
                            w_v, [jnp.full((LANES,), r0 + kk, jnp.int32)]
                        )
                        for kk in range(3)
                    ]
                    for bb in range(B):
                        for j in range(hseg):
                            src = pl.ds(bb * H3 + j * LANES, LANES)
                            acc = wv[0] * g_v[r0, src]
                            acc = acc + wv[1] * g_v[r0 + 1, src]
                            acc = acc + wv[2] * g_v[r0 + 2, src]
                            o_v[bb, pl.ds(u * H3 + j * LANES, LANES)] = acc
                    return c2

                lax.fori_loop(0, u_per_chunk, ubody, 0)
                for bb in range(B):
                    pltpu.sync_copy(o_v.at[bb], out_hbm.at[bb, chunk])

            return carry

        lax.fori_loop(0, iters, body, 0)

    return k(table, idx2d, w2d)


def _tc_mm(a, w, bias, bm, elu):
    m, kdim = a.shape
    _, n = w.shape

    def body(a_ref, w_ref, b_ref, o_ref):
        y = jnp.dot(a_ref[...], w_ref[...], preferred_element_type=jnp.float32)
        y = y + b_ref[...]
        if elu:
            y = jnp.where(y > 0.0, y, jnp.exp(jnp.minimum(y, 0.0)) - 1.0)
        o_ref[...] = y

    return pl.pallas_call(
        body,
        grid=(_cdiv(m, bm),),
        in_specs=[
            pl.BlockSpec((bm, kdim), lambda i: (i, 0)),
            pl.BlockSpec((kdim, n), lambda i: (0, 0)),
            pl.BlockSpec((1, n), lambda i: (0, 0)),
        ],
        out_specs=pl.BlockSpec((bm, n), lambda i: (i, 0)),
        out_shape=jax.ShapeDtypeStruct((m, n), jnp.float32),
    )(a, w, bias.reshape(1, n))


def _expand_w(w, l, cin, h):
    eye = jnp.eye(B, dtype=jnp.float32)
    wp = jnp.einsum("lch,bd->lbcdh", w.reshape(l, cin, h), eye)
    return wp.reshape(l * B * cin, B * h)


def kernel(x, spiral1, spiral2, spiral3, pool1_idx, pool2_idx, pool3_idx,
           pool1_w, pool2_w, pool3_w, W1, b1, W2, b2, W3, b3, Wp, bp):
    x0 = jnp.transpose(x, (1, 0, 2)).reshape(V0, B * C_IN)

    w1p = _expand_w(W1, L, C_IN, H1)
    w2p = _expand_w(W2, L, H1, H2)
    w3p = _expand_w(W3, L, H2, H3)
    b1p = jnp.tile(b1, B)
    b2p = jnp.tile(b2, B)
    b3p = jnp.tile(b3, B)

    a1 = _sc_gather(x0, spiral1.reshape(3600, 125), B * C_IN, tc_tiling=False)
    h1 = _tc_mm(a1.reshape(V0, L * B * C_IN), w1p, b1p, bm=1024, elu=True)
    p1 = _sc_pool(h1, pool1_idx.reshape(500, 75),
                  pool1_w.reshape(500, 75), 25, B * H1)

    a2 = _sc_gather(p1.reshape(V1, B * H1), spiral2.reshape(900, 125),
                    B * H1)
    h2 = _tc_mm(a2.reshape(V1, L * B * H1), w2p, b2p, bm=512, elu=True)
    p2 = _sc_pool(h2, pool2_idx.reshape(125, 75),
                  pool2_w.reshape(125, 75), 25, B * H2)

    a3 = _sc_gather(p2.reshape(V2, B * H2), spiral3.reshape(225, 125),
                    B * H2)
    h3 = _tc_mm(a3.reshape(V2, L * B * H2), w3p, b3p, bm=512, elu=True)
    qt = _sc_pool3_t(h3, pool3_idx.reshape(71, 33),
                     pool3_w.reshape(71, 33), 11)

    z = _tc_mm(qt.reshape(B, V3 * H3), Wp, bp, bm=16, elu=False)
    return z


# speedup vs baseline: 1.4232x; 1.0337x over previous
import functools

import jax
import jax.numpy as jnp
from jax import lax
from jax.experimental import pallas as pl
from jax.experimental.pallas import tpu as pltpu
from jax.experimental.pallas import tpu_sc as plsc

B = 16
V0, V1, V2, V3 = 50000, 12500, 3125, 781
L = 9
C_IN = 3
H1, H2, H3 = 16, 16, 32
LATENT = 128

NC, NS = 2, 16
NW = NC * NS
LANES = 16


def _cdiv(a, b):
    return (a + b - 1) // b


def _sc_gather(table, idx2d, d, tc_tiling=False):
    nchunk, g = idx2d.shape
    dt = table.dtype
    iters = _cdiv(nchunk, NW)
    npair = _cdiv(iters, 2)
    mesh = plsc.VectorSubcoreMesh(core_axis_name="c", subcore_axis_name="s")

    @functools.partial(
        pl.kernel,
        mesh=mesh,
        out_type=jax.ShapeDtypeStruct((nchunk, g, d), dt),
        scratch_types=[
            pltpu.VMEM((g,), jnp.int32),
            pltpu.VMEM((g,), jnp.int32),
            pltpu.VMEM((g, d), dt),
            pltpu.VMEM((g, d), dt),
            pltpu.SemaphoreType.DMA,
            pltpu.SemaphoreType.DMA,
        ],
        compiler_params=pltpu.CompilerParams(use_tc_tiling_on_sc=tc_tiling, needs_layout_passes=False),
    )
    def k(table_hbm, idx_hbm, out_hbm, idx_v0, idx_v1, rows_v0, rows_v1,
          sem0, sem1):
        wid = lax.axis_index("s") * NC + lax.axis_index("c")
        idx_b = (idx_v0, idx_v1)
        rows_b = (rows_v0, rows_v1)
        sem_b = (sem0, sem1)

        def fire(step, p):
            chunk = step * NW + wid

            @pl.when(chunk < nchunk)
            def _():
                pltpu.sync_copy(idx_hbm.at[chunk], idx_b[p])
                pltpu.make_async_copy(
                    table_hbm.at[idx_b[p]], rows_b[p], sem_b[p]
                ).start()

        def drain(step, p):
            chunk = step * NW + wid

            @pl.when(chunk < nchunk)
            def _():
                pltpu.make_async_copy(
                    table_hbm.at[idx_b[p]], rows_b[p], sem_b[p]
                ).wait()
                pltpu.sync_copy(rows_b[p], out_hbm.at[chunk])

        fire(0, 0)

        def body(t2, carry):
            t = t2 * 2
            fire(t + 1, 1)
            drain(t, 0)
            fire(t + 2, 0)
            drain(t + 1, 1)
            return carry

        lax.fori_loop(0, npair, body, 0)

    return k(table, idx2d)


def _sc_pool(table, idx2d, w2d, u_per_chunk, d, tc_tiling=False):
    nchunk, gsz = idx2d.shape
    iters = _cdiv(nchunk, NW)
    nseg = d // LANES
    mesh = plsc.VectorSubcoreMesh(core_axis_name="c", subcore_axis_name="s")

    @functools.partial(
        pl.kernel,
        mesh=mesh,
        out_type=jax.ShapeDtypeStruct((nchunk, u_per_chunk, d), jnp.float32),
        scratch_types=[
            pltpu.VMEM((gsz,), jnp.int32),
            pltpu.VMEM((gsz,), jnp.int32),
            pltpu.VMEM((gsz,), jnp.float32),
            pltpu.VMEM((gsz,), jnp.float32),
            pltpu.VMEM((gsz, d), jnp.float32),
            pltpu.VMEM((gsz, d), jnp.float32),
            pltpu.VMEM((u_per_chunk, d), jnp.float32),
            pltpu.SemaphoreType.DMA,
            pltpu.SemaphoreType.DMA,
        ],
        compiler_params=pltpu.CompilerParams(use_tc_tiling_on_sc=tc_tiling, needs_layout_passes=False),
    )
    def k(table_hbm, idx_hbm, w_hbm, out_hbm, idx_v0, idx_v1, w_v0, w_v1,
          g_v0, g_v1, o_v, sem0, sem1):
        wid = lax.axis_index("s") * NC + lax.axis_index("c")
        idx_b = (idx_v0, idx_v1)
        w_b = (w_v0, w_v1)
        g_b = (g_v0, g_v1)
        sem_b = (sem0, sem1)

        def fire(step, p):
            chunk = step * NW + wid

            @pl.when(chunk < nchunk)
            def _():
                pltpu.sync_copy(idx_hbm.at[chunk], idx_b[p])
                pltpu.sync_copy(w_hbm.at[chunk], w_b[p])
                pltpu.make_async_copy(
                    table_hbm.at[idx_b[p]], g_b[p], sem_b[p]
                ).start()

        def drain(step, p):
            chunk = step * NW + wid
            g_v, w_v = g_b[p], w_b[p]

            @pl.when(chunk < nchunk)
            def _():
                pltpu.make_async_copy(
                    table_hbm.at[idx_b[p]], g_v, sem_b[p]
                ).wait()

                def ubody(u, c2):
                    r0 = u * 3
                    wv = [
                        plsc.load_gather(
                            w_v, [jnp.full((LANES,), r0 + kk, jnp.int32)]
                        )
                        for kk in range(3)
                    ]
                    for j in range(nseg):
                        sl = pl.ds(j * LANES, LANES)
                        acc = wv[0] * g_v[r0, sl]
                        acc = acc + wv[1] * g_v[r0 + 1, sl]
                        acc = acc + wv[2] * g_v[r0 + 2, sl]
                        o_v[u, sl] = acc
                    return c2

                lax.fori_loop(0, u_per_chunk, ubody, 0)
                pltpu.sync_copy(o_v, out_hbm.at[chunk])

        fire(0, 0)

        def body(t2, carry):
            t = t2 * 2
            fire(t + 1, 1)
            drain(t, 0)
            fire(t + 2, 0)
            drain(t + 1, 1)
            return carry

        lax.fori_loop(0, _cdiv(iters, 2), body, 0)

    return k(table, idx2d, w2d)


def _sc_pool3_t(table, idx2d, w2d, u_per_chunk, tc_tiling=False):
    nchunk, gsz = idx2d.shape
    d = B * H3
    hseg = H3 // LANES
    iters = _cdiv(nchunk, NW)
    mesh = plsc.VectorSubcoreMesh(core_axis_name="c", subcore_axis_name="s")

    @functools.partial(
        pl.kernel,
        mesh=mesh,
        out_type=jax.ShapeDtypeStruct(
            (B, nchunk, u_per_chunk * H3), jnp.float32
        ),
        scratch_types=[
            pltpu.VMEM((gsz,), jnp.int32),
            pltpu.VMEM((gsz,), jnp.float32),
            pltpu.VMEM((gsz, d), jnp.float32),
            pltpu.VMEM((B, u_per_chunk * H3), jnp.float32),
            pltpu.SemaphoreType.DMA,
        ],
        compiler_params=pltpu.CompilerParams(use_tc_tiling_on_sc=tc_tiling, needs_layout_passes=False),
    )
    def k(table_hbm, idx_hbm, w_hbm, out_hbm, idx_v, w_v, g_v, o_v, sem):
        wid = lax.axis_index("s") * NC + lax.axis_index("c")

        def body(t, carry):
            chunk = t * NW + wid

            @pl.when(chunk < nchunk)
            def _():
                pltpu.sync_copy(idx_hbm.at[chunk], idx_v)
                pltpu.sync_copy(w_hbm.at[chunk], w_v)
                pltpu.async_copy(table_hbm.at[idx_v], g_v, sem).wait()

                def ubody(u, c2):
                    r0 = u * 3
                    wv = [
                        plsc.load_gather(
                            w_v, [jnp.full((LANES,), r0 + kk, jnp.int32)]
                        )
                        for kk in range(3)
                    ]
                    for bb in range(B):
                        for j in range(hseg):
                            src = pl.ds(bb * H3 + j * LANES, LANES)
                            acc = wv[0] * g_v[r0, src]
                            acc = acc + wv[1] * g_v[r0 + 1, src]
                            acc = acc + wv[2] * g_v[r0 + 2, src]
                            o_v[bb, pl.ds(u * H3 + j * LANES, LANES)] = acc
                    return c2

                lax.fori_loop(0, u_per_chunk, ubody, 0)
                for bb in range(B):
                    pltpu.sync_copy(o_v.at[bb], out_hbm.at[bb, chunk])

            return carry

        lax.fori_loop(0, iters, body, 0)

    return k(table, idx2d, w2d)


def _tc_mm(a, w, bias, bm, elu):
    m, kdim = a.shape
    _, n = w.shape

    def body(a_ref, w_ref, b_ref, o_ref):
        y = jnp.dot(a_ref[...], w_ref[...], preferred_element_type=jnp.float32)
        y = y + b_ref[...]
        if elu:
            y = jnp.where(y > 0.0, y, jnp.exp(jnp.minimum(y, 0.0)) - 1.0)
        o_ref[...] = y

    return pl.pallas_call(
        body,
        grid=(_cdiv(m, bm),),
        in_specs=[
            pl.BlockSpec((bm, kdim), lambda i: (i, 0)),
            pl.BlockSpec((kdim, n), lambda i: (0, 0)),
            pl.BlockSpec((1, n), lambda i: (0, 0)),
        ],
        out_specs=pl.BlockSpec((bm, n), lambda i: (i, 0)),
        out_shape=jax.ShapeDtypeStruct((m, n), jnp.float32),
    )(a, w, bias.reshape(1, n))


def _expand_w(w, l, cin, h):
    eye = jnp.eye(B, dtype=jnp.float32)
    wp = jnp.einsum("lch,bd->lbcdh", w.reshape(l, cin, h), eye)
    return wp.reshape(l * B * cin, B * h)


def kernel(x, spiral1, spiral2, spiral3, pool1_idx, pool2_idx, pool3_idx,
           pool1_w, pool2_w, pool3_w, W1, b1, W2, b2, W3, b3, Wp, bp):
    x0 = jnp.transpose(x, (1, 0, 2)).reshape(V0, B * C_IN)

    w1p = _expand_w(W1, L, C_IN, H1)
    w2p = _expand_w(W2, L, H1, H2)
    w3p = _expand_w(W3, L, H2, H3)
    b1p = jnp.tile(b1, B)
    b2p = jnp.tile(b2, B)
    b3p = jnp.tile(b3, B)

    a1 = _sc_gather(x0, spiral1.reshape(1800, 250), B * C_IN, tc_tiling=False)
    h1 = _tc_mm(a1.reshape(V0, L * B * C_IN), w1p, b1p, bm=1024, elu=True)
    p1 = _sc_pool(h1, pool1_idx.reshape(500, 75),
                  pool1_w.reshape(500, 75), 25, B * H1)

    a2 = _sc_gather(p1.reshape(V1, B * H1), spiral2.reshape(900, 125),
                    B * H1)
    h2 = _tc_mm(a2.reshape(V1, L * B * H1), w2p, b2p, bm=512, elu=True)
    p2 = _sc_pool(h2, pool2_idx.reshape(125, 75),
                  pool2_w.reshape(125, 75), 25, B * H2)

    a3 = _sc_gather(p2.reshape(V2, B * H2), spiral3.reshape(225, 125),
                    B * H2)
    h3 = _tc_mm(a3.reshape(V2, L * B * H2), w3p, b3p, bm=512, elu=True)
    qt = _sc_pool3_t(h3, pool3_idx.reshape(71, 33),
                     pool3_w.reshape(71, 33), 11)

    z = _tc_mm(qt.reshape(B, V3 * H3), Wp, bp, bm=16, elu=False)
    return z
